# baseline (speedup 1.0000x reference)
import functools

import jax
import jax.numpy as jnp
from jax import lax
from jax.experimental import pallas as pl
from jax.experimental.pallas import tpu as pltpu
from jax.experimental.pallas import tpu_sc as plsc

BATCH = 4096
SEQ_LEN = 200
EMBEDDING_DIM = 32

NUM_WORKERS = 32
BLK_S = 4
N_BLK = SEQ_LEN // BLK_S
LANES = 16
D_TILES = EMBEDDING_DIM // 8


def _gather_kernel(idxT_hbm, table_hbm, out_hbm, idx_v, grab_v, out_v,
                   g0, g1, w0, w1):
    gsems = (g0, g1)
    wsems = (w0, w1)
    c = lax.axis_index("c")
    s = lax.axis_index("s")
    wid = s * 2 + c
    col0 = wid * 128

    def issue_gathers(blk, b):
        pltpu.sync_copy(
            idxT_hbm.at[pl.ds(blk * BLK_S, BLK_S), pl.ds(col0, 128)],
            idx_v.at[b],
        )
        for q in range(BLK_S):
            pltpu.async_copy(
                table_hbm.at[idx_v.at[b, q]],
                grab_v.at[b, q],
                gsems[b],
            )

    def drain_gathers(b):
        for q in range(BLK_S):
            pltpu.make_async_copy(
                table_hbm.at[pl.ds(0, 128)],
                grab_v.at[b, q],
                gsems[b],
            ).wait()

    def transpose_block(b):
        rows_g = [lax.iota(jnp.int32, LANES) + g * LANES for g in range(8)]
        for q in range(BLK_S):
            src = grab_v.at[b, q]
            for dh in range(D_TILES):
                for dl in range(8):
                    col = jnp.full((LANES,), dh * 8 + dl, jnp.int32)
                    for g in range(8):
                        vec = plsc.load_gather(src, [rows_g[g], col])
                        out_v[b, dh, q, dl, pl.ds(g * LANES, LANES)] = vec

    def issue_writes(blk, b):
        for dh in range(D_TILES):
            pltpu.async_copy(
                out_v.at[b, dh],
                out_hbm.at[pl.ds(blk * BLK_S, BLK_S), dh, wid],
                wsems[b],
            )

    def drain_writes(b):
        for dh in range(D_TILES):
            pltpu.make_async_copy(
                out_hbm.at[pl.ds(0, BLK_S), dh, 0],
                out_v.at[b, dh],
                wsems[b],
            ).wait()

    issue_gathers(0, 0)
    issue_gathers(1, 1)

    def body(outer, carry):
        def half(blk, b):
            drain_gathers(b)

            @pl.when(outer >= 1)
            def _():
                drain_writes(b)

            transpose_block(b)
            issue_writes(blk, b)

            @pl.when(blk + 2 < N_BLK)
            def _():
                issue_gathers(blk + 2, b)

        half(2 * outer, 0)
        half(2 * outer + 1, 1)
        return carry

    lax.fori_loop(0, N_BLK // 2, body, 0)
    drain_writes(0)
    drain_writes(1)


def kernel(input, weight):
    idxT = input.T
    mesh = plsc.VectorSubcoreMesh(core_axis_name="c", subcore_axis_name="s")
    run = functools.partial(
        pl.kernel,
        mesh=mesh,
        out_type=jax.ShapeDtypeStruct((SEQ_LEN, D_TILES, 32, 8, 128), jnp.float32),
        scratch_types=[
            pltpu.VMEM((2, BLK_S, 128), jnp.int32),
            pltpu.VMEM((2, BLK_S, 128, EMBEDDING_DIM), jnp.float32),
            pltpu.VMEM((2, D_TILES, BLK_S, 8, 128), jnp.float32),
            pltpu.SemaphoreType.DMA,
            pltpu.SemaphoreType.DMA,
            pltpu.SemaphoreType.DMA,
            pltpu.SemaphoreType.DMA,
        ],
        compiler_params=pltpu.CompilerParams(
            use_tc_tiling_on_sc=False, needs_layout_passes=False
        ),
    )(_gather_kernel)
    out5d = run(idxT, weight)
    t = out5d.transpose(2, 4, 0, 1, 3)
    return t.reshape(BATCH, SEQ_LEN, EMBEDDING_DIM)


# speedup vs baseline: 1.4224x; 1.4224x over previous
import functools

import jax
import jax.numpy as jnp
from jax import lax
from jax.experimental import pallas as pl
from jax.experimental.pallas import tpu as pltpu
from jax.experimental.pallas import tpu_sc as plsc

BATCH = 4096
SEQ_LEN = 200
EMBEDDING_DIM = 32

NUM_WORKERS = 32
BLK_S = 4
N_BLK = SEQ_LEN // BLK_S
LANES = 16
D_TILES = EMBEDDING_DIM // 8


def _gather_kernel(idxT_hbm, table_hbm, out_hbm, idx_v, grab_v, out_v,
                   g0, g1, w0, w1):
    gsems = (g0, g1)
    wsems = (w0, w1)
    c = lax.axis_index("c")
    s = lax.axis_index("s")
    wid = s * 2 + c
    col0 = wid * 128

    def issue_gathers(blk, b):
        pltpu.sync_copy(
            idxT_hbm.at[pl.ds(blk * BLK_S, BLK_S), pl.ds(col0, 128)],
            idx_v.at[b],
        )
        for q in range(BLK_S):
            pltpu.async_copy(
                table_hbm.at[idx_v.at[b, q]],
                grab_v.at[b, q],
                gsems[b],
            )

    def drain_gathers(b):
        for q in range(BLK_S):
            pltpu.make_async_copy(
                table_hbm.at[pl.ds(0, 128)],
                grab_v.at[b, q],
                gsems[b],
            ).wait()

    def transpose_block(b):
        rows_g = [lax.iota(jnp.int32, LANES) + g * LANES for g in range(8)]
        src = grab_v.at[b]

        @plsc.parallel_loop(0, BLK_S * EMBEDDING_DIM, 1, unroll=2)
        def _(i):
            q = i // EMBEDDING_DIM
            d = i % EMBEDDING_DIM
            dh = d // 8
            dl = d % 8
            qv = jnp.zeros((LANES,), jnp.int32) + q
            col = jnp.zeros((LANES,), jnp.int32) + d
            r = (dh * BLK_S + q) * 8 + dl
            for g in range(8):
                vec = plsc.load_gather(src, [qv, rows_g[g], col])
                out_v[b, r, pl.ds(g * LANES, LANES)] = vec

    def issue_writes(blk, b):
        for dh in range(D_TILES):
            for q in range(BLK_S):
                pltpu.async_copy(
                    out_v.at[b, pl.ds((dh * BLK_S + q) * 8, 8)],
                    out_hbm.at[blk * BLK_S + q, dh, wid],
                    wsems[b],
                )

    def drain_writes(b):
        for dh in range(D_TILES):
            for q in range(BLK_S):
                pltpu.make_async_copy(
                    out_hbm.at[0, dh, 0],
                    out_v.at[b, pl.ds((dh * BLK_S + q) * 8, 8)],
                    wsems[b],
                ).wait()

    issue_gathers(0, 0)
    issue_gathers(1, 1)

    def body(outer, carry):
        def half(blk, b):
            drain_gathers(b)

            @pl.when(outer >= 1)
            def _():
                drain_writes(b)

            transpose_block(b)
            issue_writes(blk, b)

            @pl.when(blk + 2 < N_BLK)
            def _():
                issue_gathers(blk + 2, b)

        half(2 * outer, 0)
        half(2 * outer + 1, 1)
        return carry

    lax.fori_loop(0, N_BLK // 2, body, 0)
    drain_writes(0)
    drain_writes(1)


def kernel(input, weight):
    idxT = input.T
    mesh = plsc.VectorSubcoreMesh(core_axis_name="c", subcore_axis_name="s")
    run = functools.partial(
        pl.kernel,
        mesh=mesh,
        out_type=jax.ShapeDtypeStruct((SEQ_LEN, D_TILES, 32, 8, 128), jnp.float32),
        scratch_types=[
            pltpu.VMEM((2, BLK_S, 128), jnp.int32),
            pltpu.VMEM((2, BLK_S, 128, EMBEDDING_DIM), jnp.float32),
            pltpu.VMEM((2, D_TILES * BLK_S * 8, 128), jnp.float32),
            pltpu.SemaphoreType.DMA,
            pltpu.SemaphoreType.DMA,
            pltpu.SemaphoreType.DMA,
            pltpu.SemaphoreType.DMA,
        ],
        compiler_params=pltpu.CompilerParams(
            use_tc_tiling_on_sc=False, needs_layout_passes=False
        ),
    )(_gather_kernel)
    out5d = run(idxT, weight)
    t = out5d.transpose(2, 4, 0, 1, 3)
    return t.reshape(BATCH, SEQ_LEN, EMBEDDING_DIM)


# speedup vs baseline: 1.4226x; 1.0001x over previous
import functools

import jax
import jax.numpy as jnp
from jax import lax
from jax.experimental import pallas as pl
from jax.experimental.pallas import tpu as pltpu
from jax.experimental.pallas import tpu_sc as plsc

BATCH = 4096
SEQ_LEN = 200
EMBEDDING_DIM = 32

NUM_WORKERS = 32
BLK_S = 4
N_BLK = SEQ_LEN // BLK_S
LANES = 16
D_TILES = EMBEDDING_DIM // 8
R16 = D_TILES * BLK_S


def _gather_kernel(idxT_hbm, table_hbm, out_hbm, idx_v, grab_v, out_v,
                   g0, g1, w0, w1):
    gsems = (g0, g1)
    wsems = (w0, w1)
    c = lax.axis_index("c")
    s = lax.axis_index("s")
    wid = s * 2 + c
    col0 = wid * 128

    def issue_gathers(blk, b):
        pltpu.sync_copy(
            idxT_hbm.at[pl.ds(blk * BLK_S, BLK_S), pl.ds(col0, 128)],
            idx_v.at[b],
        )
        for q in range(BLK_S):
            pltpu.async_copy(
                table_hbm.at[idx_v.at[b, q]],
                grab_v.at[b, pl.ds(q * 128, 128)],
                gsems[b],
            )

    def drain_gathers(b):
        pltpu.make_async_copy(
            table_hbm.at[pl.ds(0, BLK_S * 128)],
            grab_v.at[b],
            gsems[b],
        ).wait()

    def transpose_block(b):
        rows_qg = [
            lax.iota(jnp.int32, LANES) + (q * 128 + g * LANES)
            for q in range(BLK_S)
            for g in range(8)
        ]
        src = grab_v.at[b]

        @plsc.parallel_loop(0, EMBEDDING_DIM, 1, unroll=2)
        def _(d):
            dh = d // 8
            dl = d % 8
            col = jnp.zeros((LANES,), jnp.int32) + d
            r_base = dh * BLK_S
            for q in range(BLK_S):
                for g in range(8):
                    vec = plsc.load_gather(src, [rows_qg[q * 8 + g], col])
                    out_v[b, r_base + q, dl, pl.ds(g * LANES, LANES)] = vec

    def issue_writes(blk, b):
        for dh in range(D_TILES):
            pltpu.async_copy(
                out_v.at[b, pl.ds(dh * BLK_S, BLK_S)],
                out_hbm.at[pl.ds(blk * BLK_S, BLK_S), dh, wid],
                wsems[b],
            )

    def drain_writes(b):
        pltpu.make_async_copy(
            out_hbm.at[pl.ds(0, R16), 0, 0],
            out_v.at[b],
            wsems[b],
        ).wait()

    issue_gathers(0, 0)
    issue_gathers(1, 1)

    def body(outer, carry):
        def half(blk, b):
            drain_gathers(b)

            @pl.when(outer >= 1)
            def _():
                drain_writes(b)

            transpose_block(b)
            issue_writes(blk, b)

            @pl.when(blk + 2 < N_BLK)
            def _():
                issue_gathers(blk + 2, b)

        half(2 * outer, 0)
        half(2 * outer + 1, 1)
        return carry

    lax.fori_loop(0, N_BLK // 2, body, 0)
    drain_writes(0)
    drain_writes(1)


def kernel(input, weight):
    idxT = input.T
    mesh = plsc.VectorSubcoreMesh(core_axis_name="c", subcore_axis_name="s")
    run = functools.partial(
        pl.kernel,
        mesh=mesh,
        out_type=jax.ShapeDtypeStruct((SEQ_LEN, D_TILES, 32, 8, 128), jnp.float32),
        scratch_types=[
            pltpu.VMEM((2, BLK_S, 128), jnp.int32),
            pltpu.VMEM((2, BLK_S * 128, EMBEDDING_DIM), jnp.float32),
            pltpu.VMEM((2, R16, 8, 128), jnp.float32),
            pltpu.SemaphoreType.DMA,
            pltpu.SemaphoreType.DMA,
            pltpu.SemaphoreType.DMA,
            pltpu.SemaphoreType.DMA,
        ],
        compiler_params=pltpu.CompilerParams(
            use_tc_tiling_on_sc=False, needs_layout_passes=False
        ),
    )(_gather_kernel)
    out5d = run(idxT, weight)
    t = out5d.transpose(2, 4, 0, 1, 3)
    return t.reshape(BATCH, SEQ_LEN, EMBEDDING_DIM)


# speedup vs baseline: 1.4259x; 1.0023x over previous
import functools

import jax
import jax.numpy as jnp
from jax import lax
from jax.experimental import pallas as pl
from jax.experimental.pallas import tpu as pltpu
from jax.experimental.pallas import tpu_sc as plsc

BATCH = 4096
SEQ_LEN = 200
EMBEDDING_DIM = 32

NUM_WORKERS = 32
BLK_S = 5
N_BLK = SEQ_LEN // BLK_S
LANES = 16
D_TILES = EMBEDDING_DIM // 8
R16 = D_TILES * BLK_S


def _gather_kernel(idxT_hbm, table_hbm, out_hbm, idx_v, grab_v, out_v,
                   g0, g1, w0, w1):
    gsems = (g0, g1)
    wsems = (w0, w1)
    c = lax.axis_index("c")
    s = lax.axis_index("s")
    wid = s * 2 + c
    col0 = wid * 128

    def issue_gathers(blk, b):
        pltpu.sync_copy(
            idxT_hbm.at[pl.ds(blk * BLK_S, BLK_S), pl.ds(col0, 128)],
            idx_v.at[b],
        )
        for q in range(BLK_S):
            pltpu.async_copy(
                table_hbm.at[idx_v.at[b, q]],
                grab_v.at[b, pl.ds(q * 128, 128)],
                gsems[b],
            )

    def drain_gathers(b):
        pltpu.make_async_copy(
            table_hbm.at[pl.ds(0, BLK_S * 128)],
            grab_v.at[b],
            gsems[b],
        ).wait()

    def transpose_block(b):
        rows_qg = [
            lax.iota(jnp.int32, LANES) + (q * 128 + g * LANES)
            for q in range(BLK_S)
            for g in range(8)
        ]
        src = grab_v.at[b]

        @plsc.parallel_loop(0, EMBEDDING_DIM, 1, unroll=4)
        def _(d):
            dh = d // 8
            dl = d % 8
            col = jnp.zeros((LANES,), jnp.int32) + d
            r_base = dh * BLK_S
            for q in range(BLK_S):
                for g in range(8):
                    vec = plsc.load_gather(src, [rows_qg[q * 8 + g], col])
                    out_v[b, r_base + q, dl, pl.ds(g * LANES, LANES)] = vec

    def issue_writes(blk, b):
        for dh in range(D_TILES):
            pltpu.async_copy(
                out_v.at[b, pl.ds(dh * BLK_S, BLK_S)],
                out_hbm.at[pl.ds(blk * BLK_S, BLK_S), dh, wid],
                wsems[b],
            )

    def drain_writes(b):
        pltpu.make_async_copy(
            out_hbm.at[pl.ds(0, R16), 0, 0],
            out_v.at[b],
            wsems[b],
        ).wait()

    issue_gathers(0, 0)
    issue_gathers(1, 1)

    def body(outer, carry):
        def half(blk, b):
            drain_gathers(b)

            @pl.when(outer >= 1)
            def _():
                drain_writes(b)

            transpose_block(b)
            issue_writes(blk, b)

            @pl.when(blk + 2 < N_BLK)
            def _():
                issue_gathers(blk + 2, b)

        half(2 * outer, 0)
        half(2 * outer + 1, 1)
        return carry

    lax.fori_loop(0, N_BLK // 2, body, 0)
    drain_writes(0)
    drain_writes(1)


def kernel(input, weight):
    idxT = input.T
    mesh = plsc.VectorSubcoreMesh(core_axis_name="c", subcore_axis_name="s")
    run = functools.partial(
        pl.kernel,
        mesh=mesh,
        out_type=jax.ShapeDtypeStruct((SEQ_LEN, D_TILES, 32, 8, 128), jnp.float32),
        scratch_types=[
            pltpu.VMEM((2, BLK_S, 128), jnp.int32),
            pltpu.VMEM((2, BLK_S * 128, EMBEDDING_DIM), jnp.float32),
            pltpu.VMEM((2, R16, 8, 128), jnp.float32),
            pltpu.SemaphoreType.DMA,
            pltpu.SemaphoreType.DMA,
            pltpu.SemaphoreType.DMA,
            pltpu.SemaphoreType.DMA,
        ],
        compiler_params=pltpu.CompilerParams(
            use_tc_tiling_on_sc=False, needs_layout_passes=False
        ),
    )(_gather_kernel)
    out5d = run(idxT, weight)
    t = out5d.transpose(2, 4, 0, 1, 3)
    return t.reshape(BATCH, SEQ_LEN, EMBEDDING_DIM)
